# baseline (speedup 1.0000x reference)
import jax
import jax.numpy as jnp
from jax import lax
from jax.experimental import pallas as pl
from jax.experimental.pallas import tpu as pltpu
from jax.experimental.pallas import tpu_sc as plsc

_NROW = 100000
_D = 64
_NC = 2
_NS = 16
_NW = _NC * _NS
_CH = 400
_CPT = _NROW // _CH
_KPT = 8
_K = 2 * _KPT
_NBUF = 2
_EXTRA = _CPT - (_KPT - 1) * _NW


def _sc_body(u_hbm, i_hbm, oi_hbm, ou_hbm, bufs, in_sems, out_sems):
    wid = lax.axis_index("s") * _NC + lax.axis_index("c")
    has_extra = wid < _EXTRA

    def loc(k):
        t = k // _KPT
        j = k % _KPT
        r = (j * _NW + wid) * _CH
        r = jnp.minimum(r, _NROW - _CH)
        return t, r

    def static_valid(k):
        return (k % _KPT) != (_KPT - 1)

    def in_cp(k, slot):
        t, r = loc(k)
        src = (i_hbm, u_hbm)[t]
        return pltpu.make_async_copy(
            src.at[pl.ds(r, _CH), :], bufs.at[slot], in_sems.at[slot]
        )

    def out_cp(k, slot):
        t, r = loc(k)
        dst = (oi_hbm, ou_hbm)[t]
        return pltpu.make_async_copy(
            bufs.at[slot], dst.at[pl.ds(r, _CH), :], out_sems.at[slot]
        )

    def guarded(k, fn):
        if static_valid(k):
            fn()
        else:
            @pl.when(has_extra)
            def _():
                fn()

    in_cp(0, 0).start()
    for k in range(_K):
        s = k % _NBUF
        nk = k + 1
        if nk < _K:
            ns = nk % _NBUF
            if nk >= _NBUF:
                guarded(nk - _NBUF, out_cp(nk - _NBUF, ns).wait)
            guarded(nk, in_cp(nk, ns).start)
        guarded(k, in_cp(k, s).wait)
        guarded(k, out_cp(k, s).start)
    for k in range(_K - _NBUF, _K):
        guarded(k, out_cp(k, k % _NBUF).wait)


def kernel(embed_user, embed_item):
    out_type = (
        jax.ShapeDtypeStruct(embed_item.shape, embed_item.dtype),
        jax.ShapeDtypeStruct(embed_user.shape, embed_user.dtype),
    )
    f = pl.kernel(
        _sc_body,
        out_type=out_type,
        mesh=plsc.VectorSubcoreMesh(core_axis_name="c", subcore_axis_name="s"),
        scratch_types=[
            pltpu.VMEM((_NBUF, _CH, _D), jnp.float32),
            pltpu.SemaphoreType.DMA((_NBUF,)),
            pltpu.SemaphoreType.DMA((_NBUF,)),
        ],
    )
    return f(embed_user, embed_item)


# speedup vs baseline: 1.0008x; 1.0008x over previous
import jax
import jax.numpy as jnp
from jax import lax
from jax.experimental import pallas as pl
from jax.experimental.pallas import tpu as pltpu
from jax.experimental.pallas import tpu_sc as plsc

_NROW = 100000
_D = 64
_NC = 2
_NS = 16
_NW = _NC * _NS
_CH = 400
_CPT = _NROW // _CH
_KPT = 8
_K = 2 * _KPT
_NBUF = 2
_EXTRA = _CPT - (_KPT - 1) * _NW


def _sc_body(u_hbm, i_hbm, oi_hbm, ou_hbm, bufs, in_sems, out_sems):
    wid = lax.axis_index("s") * _NC + lax.axis_index("c")
    has_extra = wid < _EXTRA

    def loc(k):
        t = k // _KPT
        j = k % _KPT
        r = (j * _NW + wid) * _CH
        r = jnp.minimum(r, _NROW - _CH)
        return t, r

    def static_valid(k):
        return (k % _KPT) != (_KPT - 1)

    def in_cp(k, slot):
        t, r = loc(k)
        src = (i_hbm, u_hbm)[t]
        return pltpu.make_async_copy(
            src.at[pl.ds(r, _CH), :], bufs.at[slot], in_sems.at[slot]
        )

    def out_cp(k, slot):
        t, r = loc(k)
        dst = (oi_hbm, ou_hbm)[t]
        return pltpu.make_async_copy(
            bufs.at[slot], dst.at[pl.ds(r, _CH), :], out_sems.at[slot]
        )

    def guarded(k, fn):
        if static_valid(k):
            fn()
        else:
            @pl.when(has_extra)
            def _():
                fn()

    in_cp(0, 0).start()
    for k in range(_K):
        s = k % _NBUF
        nk = k + 1
        if nk < _K:
            ns = nk % _NBUF
            if nk >= _NBUF:
                guarded(nk - _NBUF, out_cp(nk - _NBUF, ns).wait)
            guarded(nk, in_cp(nk, ns).start)
        guarded(k, in_cp(k, s).wait)
        guarded(k, out_cp(k, s).start)
    for k in range(_K - _NBUF, _K):
        guarded(k, out_cp(k, k % _NBUF).wait)


def kernel(embed_user, embed_item):
    out_type = (
        jax.ShapeDtypeStruct(embed_item.shape, embed_item.dtype),
        jax.ShapeDtypeStruct(embed_user.shape, embed_user.dtype),
    )
    f = pl.kernel(
        _sc_body,
        out_type=out_type,
        mesh=plsc.VectorSubcoreMesh(core_axis_name="c", subcore_axis_name="s"),
        scratch_types=[
            pltpu.VMEM((_NBUF, _CH, _D), jnp.float32),
            pltpu.SemaphoreType.DMA((_NBUF,)),
            pltpu.SemaphoreType.DMA((_NBUF,)),
        ],
        compiler_params=pltpu.CompilerParams(use_tc_tiling_on_sc=True),
    )
    return f(embed_user, embed_item)
